# Initial kernel scaffold; baseline (speedup 1.0000x reference)
import jax
import jax.numpy as jnp
from jax.experimental import pallas as pl


def kernel(x, edge_index, W_pre, W_nn, bias):
    raise NotImplementedError("write your pallas kernel here")


# speedup vs baseline: 5.4803x; 5.4803x over previous
import functools

import jax
import jax.numpy as jnp
from jax import lax
from jax.experimental import pallas as pl
from jax.experimental.pallas import tpu as pltpu
from jax.experimental.pallas import tpu_sc as plsc

_NC = 2
_NS = 16
_NW = _NC * _NS
_NPW = 320
_CPC = 4
_NCH = _NPW // _CPC
_N_PAD = _NW * _NPW
_K = 32
_C = 128
_LC = _C // 16


def _pre_body(x_ref, w_ref, o_ref):
    h = lax.dot_general(x_ref[...], w_ref[...], (((0,), (1,)), ((), ())),
                        preferred_element_type=jnp.float32)
    o_ref[...] = jnp.maximum(h, 0.0)


def _post_body(x_ref, at_ref, wx_ref, wa_ref, b_ref, o_ref):
    t = lax.dot_general(wx_ref[...], x_ref[...], (((1,), (0,)), ((), ())),
                        preferred_element_type=jnp.float32)
    t = t + lax.dot_general(wa_ref[...], at_ref[...], (((1,), (1,)), ((), ())),
                            preferred_element_type=jnp.float32)
    t = jnp.maximum(t, 0.0) + b_ref[...]
    nrm = jnp.sqrt(jnp.sum(t * t, axis=0, keepdims=True))
    o_ref[...] = t / jnp.maximum(nrm, 1e-12)


def _sc_gather_max(ht, idx3):
    mesh = plsc.VectorSubcoreMesh(core_axis_name="c", subcore_axis_name="s")

    @functools.partial(
        pl.kernel, mesh=mesh,
        out_type=jax.ShapeDtypeStruct((_N_PAD, _C), jnp.float32),
        scratch_types=[
            pltpu.VMEM((_NCH, 128), jnp.int32),
            pltpu.VMEM((128, _C), jnp.float32),
            pltpu.VMEM((_NPW, _C), jnp.float32),
            pltpu.SemaphoreType.DMA,
        ],
    )
    def k(ht_hbm, idx_hbm, out_hbm, idx_v, rows_v, out_v, sem):
        wid = lax.axis_index("s") * _NC + lax.axis_index("c")
        pltpu.sync_copy(idx_hbm.at[wid], idx_v)

        def chunk_body(c, carry):
            pltpu.async_copy(ht_hbm.at[idx_v.at[c]], rows_v, sem).wait()
            for i in range(_CPC):
                accs = tuple(rows_v[i * _K, pl.ds(d * 16, 16)]
                             for d in range(_LC))

                def kbody(kk, a, i=i):
                    return tuple(
                        jnp.maximum(a[d], rows_v[i * _K + kk, pl.ds(d * 16, 16)])
                        for d in range(_LC))

                accs = lax.fori_loop(1, _K, kbody, accs)
                for d in range(_LC):
                    out_v[c * _CPC + i, pl.ds(d * 16, 16)] = accs[d]
            return carry

        lax.fori_loop(0, _NCH, chunk_body, 0)
        pltpu.sync_copy(out_v, out_hbm.at[pl.ds(wid * _NPW, _NPW)])

    return k(ht, idx3)


def kernel(x, edge_index, W_pre, W_nn, bias):
    B, C_in, N, _ = x.shape
    C_out = W_pre.shape[0]
    assert (B, C_in, C_out, edge_index.shape[-1]) == (1, _C, _C, _K)

    X = x[0, :, :, 0]
    idx = edge_index[0, 0]
    pad = _N_PAD - N
    Xp = jnp.pad(X, ((0, 0), (0, pad)))
    idxp = jnp.pad(idx, ((0, pad), (0, 0)))
    idx3 = idxp.reshape(_NW, _NCH, 128)

    ht = pl.pallas_call(
        _pre_body,
        out_shape=jax.ShapeDtypeStruct((_N_PAD, C_out), jnp.float32),
    )(Xp, W_pre)

    aggr = _sc_gather_max(ht, idx3)

    Wx = W_nn[:, :C_in]
    Wa = W_nn[:, C_in:]
    b2 = bias.reshape(C_out, 1)
    o = pl.pallas_call(
        _post_body,
        out_shape=jax.ShapeDtypeStruct((C_out, _N_PAD), jnp.float32),
    )(Xp, aggr, Wx, Wa, b2)

    return o[:, :N].reshape(1, C_out, N, 1)


# speedup vs baseline: 6.2363x; 1.1379x over previous
import functools

import jax
import jax.numpy as jnp
from jax import lax
from jax.experimental import pallas as pl
from jax.experimental.pallas import tpu as pltpu
from jax.experimental.pallas import tpu_sc as plsc

_NC = 2
_NS = 16
_NW = _NC * _NS
_NPW = 320
_CPC = 4
_NCH = _NPW // _CPC
_N_PAD = _NW * _NPW
_K = 32
_C = 128
_LC = _C // 16


def _pre_body(x_ref, w_ref, o_ref):
    h = lax.dot_general(x_ref[...], w_ref[...], (((0,), (1,)), ((), ())),
                        preferred_element_type=jnp.float32)
    o_ref[...] = jnp.maximum(h, 0.0)


def _post_body(x_ref, at_ref, wx_ref, wa_ref, b_ref, o_ref):
    t = lax.dot_general(wx_ref[...], x_ref[...], (((1,), (0,)), ((), ())),
                        preferred_element_type=jnp.float32)
    t = t + lax.dot_general(wa_ref[...], at_ref[...], (((1,), (1,)), ((), ())),
                            preferred_element_type=jnp.float32)
    t = jnp.maximum(t, 0.0) + b_ref[...]
    nrm = jnp.sqrt(jnp.sum(t * t, axis=0, keepdims=True))
    o_ref[...] = t / jnp.maximum(nrm, 1e-12)


def _sc_gather_max(ht, idx3):
    mesh = plsc.VectorSubcoreMesh(core_axis_name="c", subcore_axis_name="s")

    @functools.partial(
        pl.kernel, mesh=mesh,
        out_type=jax.ShapeDtypeStruct((_N_PAD, _C), jnp.float32),
        scratch_types=[
            pltpu.VMEM((_NCH, 128), jnp.int32),
            pltpu.VMEM((2, 128, _C), jnp.float32),
            pltpu.VMEM((_NPW, _C), jnp.float32),
            pltpu.SemaphoreType.DMA,
            pltpu.SemaphoreType.DMA,
        ],
    )
    def k(ht_hbm, idx_hbm, out_hbm, idx_v, rows_v, out_v, sem0, sem1):
        wid = lax.axis_index("s") * _NC + lax.axis_index("c")
        pltpu.sync_copy(idx_hbm.at[wid], idx_v)
        sems = (sem0, sem1)
        for b in range(2):
            pltpu.async_copy(ht_hbm.at[idx_v.at[b]], rows_v.at[b], sems[b])

        def outer(c2, carry):
            for b in range(2):
                c = c2 * 2 + b
                pltpu.make_async_copy(
                    ht_hbm.at[idx_v.at[c]], rows_v.at[b], sems[b]).wait()
                for i in range(_CPC):
                    accs = tuple(rows_v[b, i * _K, pl.ds(d * 16, 16)]
                                 for d in range(_LC))

                    def kbody(kk, a, b=b, i=i):
                        return tuple(
                            jnp.maximum(
                                a[d], rows_v[b, i * _K + kk, pl.ds(d * 16, 16)])
                            for d in range(_LC))

                    accs = lax.fori_loop(1, _K, kbody, accs)
                    for d in range(_LC):
                        out_v[c * _CPC + i, pl.ds(d * 16, 16)] = accs[d]

                @pl.when(c + 2 < _NCH)
                def _(b=b, c=c):
                    pltpu.async_copy(
                        ht_hbm.at[idx_v.at[c + 2]], rows_v.at[b], sems[b])
            return carry

        lax.fori_loop(0, _NCH // 2, outer, 0)
        pltpu.sync_copy(out_v, out_hbm.at[pl.ds(wid * _NPW, _NPW)])

    return k(ht, idx3)


def kernel(x, edge_index, W_pre, W_nn, bias):
    B, C_in, N, _ = x.shape
    C_out = W_pre.shape[0]
    assert (B, C_in, C_out, edge_index.shape[-1]) == (1, _C, _C, _K)

    X = x[0, :, :, 0]
    idx = edge_index[0, 0]
    pad = _N_PAD - N
    Xp = jnp.pad(X, ((0, 0), (0, pad)))
    idxp = jnp.pad(idx, ((0, pad), (0, 0)))
    idx3 = idxp.reshape(_NW, _NCH, 128)

    ht = pl.pallas_call(
        _pre_body,
        out_shape=jax.ShapeDtypeStruct((_N_PAD, C_out), jnp.float32),
    )(Xp, W_pre)

    aggr = _sc_gather_max(ht, idx3)

    Wx = W_nn[:, :C_in]
    Wa = W_nn[:, C_in:]
    b2 = bias.reshape(C_out, 1)
    o = pl.pallas_call(
        _post_body,
        out_shape=jax.ShapeDtypeStruct((C_out, _N_PAD), jnp.float32),
    )(Xp, aggr, Wx, Wa, b2)

    return o[:, :N].reshape(1, C_out, N, 1)


# speedup vs baseline: 8.0747x; 1.2948x over previous
import functools

import jax
import jax.numpy as jnp
from jax import lax
from jax.experimental import pallas as pl
from jax.experimental.pallas import tpu as pltpu
from jax.experimental.pallas import tpu_sc as plsc

_NC = 2
_NS = 16
_NW = _NC * _NS
_NPW = 320
_CPC = 4
_NCH = _NPW // _CPC
_N_PAD = _NW * _NPW
_K = 32
_C = 128
_LC = _C // 16


def _pre_body(x_ref, w_ref, o_ref):
    h = lax.dot_general(x_ref[...], w_ref[...], (((0,), (1,)), ((), ())),
                        preferred_element_type=jnp.float32)
    o_ref[...] = jnp.maximum(h, 0.0).astype(jnp.bfloat16)


def _post_body(x_ref, at_ref, wx_ref, wa_ref, b_ref, o_ref):
    a = at_ref[...].astype(jnp.float32)
    t = lax.dot_general(wx_ref[...], x_ref[...], (((1,), (0,)), ((), ())),
                        preferred_element_type=jnp.float32)
    t = t + lax.dot_general(wa_ref[...], a, (((1,), (1,)), ((), ())),
                            preferred_element_type=jnp.float32)
    t = jnp.maximum(t, 0.0) + b_ref[...]
    nrm = jnp.sqrt(jnp.sum(t * t, axis=0, keepdims=True))
    o_ref[...] = t / jnp.maximum(nrm, 1e-12)


def _sc_gather_max(ht, idx3):
    mesh = plsc.VectorSubcoreMesh(core_axis_name="c", subcore_axis_name="s")

    w2 = _C // 2

    @functools.partial(
        pl.kernel, mesh=mesh,
        compiler_params=pltpu.CompilerParams(use_tc_tiling_on_sc=False),
        out_type=jax.ShapeDtypeStruct((_N_PAD, w2), jnp.int32),
        scratch_types=[
            pltpu.VMEM((_NCH, 128), jnp.int32),
            pltpu.VMEM((2, 128, w2), jnp.int32),
            pltpu.VMEM((_NPW, w2), jnp.int32),
            pltpu.SemaphoreType.DMA,
            pltpu.SemaphoreType.DMA,
        ],
    )
    def k(ht_hbm, idx_hbm, out_hbm, idx_v, rows_v, out_v, sem0, sem1):
        wid = lax.axis_index("s") * _NC + lax.axis_index("c")
        pltpu.sync_copy(idx_hbm.at[wid], idx_v)
        sems = (sem0, sem1)
        for b in range(2):
            pltpu.async_copy(ht_hbm.at[idx_v.at[b]], rows_v.at[b], sems[b])

        nd = w2 // 16

        def outer(c2, carry):
            for b in range(2):
                c = c2 * 2 + b
                pltpu.make_async_copy(
                    ht_hbm.at[idx_v.at[c]], rows_v.at[b], sems[b]).wait()
                for i in range(_CPC):
                    w0 = [rows_v[b, i * _K, pl.ds(d * 16, 16)]
                          for d in range(nd)]
                    los = [w << 16 for w in w0]
                    his = [w >> 16 for w in w0]
                    for kk in range(1, _K):
                        for d in range(nd):
                            w = rows_v[b, i * _K + kk, pl.ds(d * 16, 16)]
                            los[d] = jnp.maximum(los[d], w << 16)
                            his[d] = jnp.maximum(his[d], w >> 16)
                    for d in range(nd):
                        out_v[c * _CPC + i, pl.ds(d * 16, 16)] = (
                            (los[d] >> 16) | (his[d] << 16))

                @pl.when(c + 2 < _NCH)
                def _(b=b, c=c):
                    pltpu.async_copy(
                        ht_hbm.at[idx_v.at[c + 2]], rows_v.at[b], sems[b])
            return carry

        lax.fori_loop(0, _NCH // 2, outer, 0)
        pltpu.sync_copy(out_v, out_hbm.at[pl.ds(wid * _NPW, _NPW)])

    return k(ht, idx3)


def kernel(x, edge_index, W_pre, W_nn, bias):
    B, C_in, N, _ = x.shape
    C_out = W_pre.shape[0]
    assert (B, C_in, C_out, edge_index.shape[-1]) == (1, _C, _C, _K)

    X = x[0, :, :, 0]
    idx = edge_index[0, 0]
    pad = _N_PAD - N
    Xp = jnp.pad(X, ((0, 0), (0, pad)))
    idxp = jnp.pad(idx, ((0, pad), (0, 0)))
    idx3 = idxp.reshape(_NW, _NCH, 128)

    ht = pl.pallas_call(
        _pre_body,
        out_shape=jax.ShapeDtypeStruct((_N_PAD, C_out), jnp.bfloat16),
    )(Xp, W_pre)
    htp = lax.bitcast_convert_type(
        ht.reshape(_N_PAD, C_out // 2, 2), jnp.int32)

    aggr_p = _sc_gather_max(htp, idx3)
    aggr = lax.bitcast_convert_type(
        aggr_p, jnp.bfloat16).reshape(_N_PAD, C_out)

    Wx = W_nn[:, :C_in]
    Wa = W_nn[:, C_in:]
    b2 = bias.reshape(C_out, 1)
    o = pl.pallas_call(
        _post_body,
        out_shape=jax.ShapeDtypeStruct((C_out, _N_PAD), jnp.float32),
    )(Xp, aggr, Wx, Wa, b2)

    return o[:, :N].reshape(1, C_out, N, 1)


# speedup vs baseline: 8.1517x; 1.0095x over previous
import functools

import jax
import jax.numpy as jnp
from jax import lax
from jax.experimental import pallas as pl
from jax.experimental.pallas import tpu as pltpu
from jax.experimental.pallas import tpu_sc as plsc

_NC = 2
_NS = 16
_NW = _NC * _NS
_NPW = 320
_CPC = 4
_NCH = _NPW // _CPC
_N_PAD = _NW * _NPW
_K = 32
_C = 128
_LC = _C // 16


def _pre_body(x_ref, w_ref, o_ref):
    h = lax.dot_general(x_ref[...], w_ref[...], (((0,), (1,)), ((), ())),
                        preferred_element_type=jnp.float32)
    o_ref[...] = jnp.maximum(h, 0.0).astype(jnp.bfloat16)


def _post_body(x_ref, at_ref, wx_ref, wa_ref, b_ref, o_ref):
    a = at_ref[...].astype(jnp.float32)
    t = lax.dot_general(wx_ref[...], x_ref[...], (((1,), (0,)), ((), ())),
                        preferred_element_type=jnp.float32)
    t = t + lax.dot_general(wa_ref[...], a, (((1,), (1,)), ((), ())),
                            preferred_element_type=jnp.float32)
    t = jnp.maximum(t, 0.0) + b_ref[...]
    nrm = jnp.sqrt(jnp.sum(t * t, axis=0, keepdims=True))
    o_ref[...] = t / jnp.maximum(nrm, 1e-12)


def _sc_gather_max(ht, idx3):
    mesh = plsc.VectorSubcoreMesh(core_axis_name="c", subcore_axis_name="s")

    w2 = _C // 2

    @functools.partial(
        pl.kernel, mesh=mesh,
        compiler_params=pltpu.CompilerParams(use_tc_tiling_on_sc=False),
        out_type=jax.ShapeDtypeStruct((_N_PAD, w2), jnp.int32),
        scratch_types=[
            pltpu.VMEM((_NCH, 128), jnp.int32),
            pltpu.VMEM((4, 128, w2), jnp.int32),
            pltpu.VMEM((_NPW, w2), jnp.int32),
            pltpu.SemaphoreType.DMA,
            pltpu.SemaphoreType.DMA,
            pltpu.SemaphoreType.DMA,
            pltpu.SemaphoreType.DMA,
        ],
    )
    def k(ht_hbm, idx_hbm, out_hbm, idx_v, rows_v, out_v, s0, s1, s2, s3):
        wid = lax.axis_index("s") * _NC + lax.axis_index("c")
        pltpu.sync_copy(idx_hbm.at[wid], idx_v)
        sems = (s0, s1, s2, s3)
        for b in range(3):
            pltpu.async_copy(ht_hbm.at[idx_v.at[b]], rows_v.at[b], sems[b])

        nd = w2 // 16

        def outer(c4, carry):
            for b in range(4):
                c = c4 * 4 + b
                pltpu.make_async_copy(
                    ht_hbm.at[idx_v.at[c]], rows_v.at[b], sems[b]).wait()
                for i in range(_CPC):
                    base = i * _K
                    w0 = [rows_v[b, base, pl.ds(d * 16, 16)]
                          for d in range(nd)]
                    los = [w << 16 for w in w0]
                    his = [w >> 16 for w in w0]

                    def kbody(t, acc, b=b, base=base):
                        los, his = list(acc[:nd]), list(acc[nd:])
                        for u in range(4):
                            for d in range(nd):
                                w = rows_v[b, base + t * 4 + u,
                                           pl.ds(d * 16, 16)]
                                los[d] = jnp.maximum(los[d], w << 16)
                                his[d] = jnp.maximum(his[d], w >> 16)
                        return tuple(los) + tuple(his)

                    acc = lax.fori_loop(0, _K // 4, kbody,
                                        tuple(los) + tuple(his))
                    los, his = acc[:nd], acc[nd:]
                    for d in range(nd):
                        out_v[c * _CPC + i, pl.ds(d * 16, 16)] = (
                            (los[d] >> 16) | (his[d] << 16))

                nxt = (b + 3) % 4

                @pl.when(c + 3 < _NCH)
                def _(c=c, nxt=nxt):
                    pltpu.async_copy(
                        ht_hbm.at[idx_v.at[c + 3]], rows_v.at[nxt], sems[nxt])
            return carry

        lax.fori_loop(0, _NCH // 4, outer, 0)
        pltpu.sync_copy(out_v, out_hbm.at[pl.ds(wid * _NPW, _NPW)])

    return k(ht, idx3)


def kernel(x, edge_index, W_pre, W_nn, bias):
    B, C_in, N, _ = x.shape
    C_out = W_pre.shape[0]
    assert (B, C_in, C_out, edge_index.shape[-1]) == (1, _C, _C, _K)

    X = x[0, :, :, 0]
    idx = edge_index[0, 0]
    pad = _N_PAD - N
    Xp = jnp.pad(X, ((0, 0), (0, pad)))
    idxp = jnp.pad(idx, ((0, pad), (0, 0)))
    idx3 = idxp.reshape(_NW, _NCH, 128)

    ht = pl.pallas_call(
        _pre_body,
        out_shape=jax.ShapeDtypeStruct((_N_PAD, C_out), jnp.bfloat16),
    )(Xp, W_pre)
    htp = lax.bitcast_convert_type(
        ht.reshape(_N_PAD, C_out // 2, 2), jnp.int32)

    aggr_p = _sc_gather_max(htp, idx3)
    aggr = lax.bitcast_convert_type(
        aggr_p, jnp.bfloat16).reshape(_N_PAD, C_out)

    Wx = W_nn[:, :C_in]
    Wa = W_nn[:, C_in:]
    b2 = bias.reshape(C_out, 1)
    o = pl.pallas_call(
        _post_body,
        out_shape=jax.ShapeDtypeStruct((C_out, _N_PAD), jnp.float32),
    )(Xp, aggr, Wx, Wa, b2)

    return o[:, :N].reshape(1, C_out, N, 1)


# speedup vs baseline: 16.1767x; 1.9845x over previous
import functools

import jax
import jax.numpy as jnp
from jax import lax
from jax.experimental import pallas as pl
from jax.experimental.pallas import tpu as pltpu
from jax.experimental.pallas import tpu_sc as plsc

_NC = 2
_NS = 16
_N_PAD = 10240
_NH = _N_PAD // 2
_BLK = 640
_NBLK = _NH // _BLK
_K = 32
_C = 128
_W2 = _C // 2
_WPT = _W2 // _NS


def _pre_body(x_ref, w_ref, o_ref):
    h = lax.dot_general(w_ref[...], x_ref[...], (((1,), (0,)), ((), ())),
                        preferred_element_type=jnp.float32)
    o_ref[...] = jnp.maximum(h, 0.0).astype(jnp.bfloat16)


def _post_body(x_ref, a_ref, wx_ref, wa_ref, b_ref, o_ref):
    a = a_ref[...].astype(jnp.float32)
    t = lax.dot_general(wx_ref[...], x_ref[...], (((1,), (0,)), ((), ())),
                        preferred_element_type=jnp.float32)
    t = t + lax.dot_general(wa_ref[...], a, (((1,), (0,)), ((), ())),
                            preferred_element_type=jnp.float32)
    t = jnp.maximum(t, 0.0) + b_ref[...]
    nrm = jnp.sqrt(jnp.sum(t * t, axis=0, keepdims=True))
    o_ref[...] = t / jnp.maximum(nrm, 1e-12)


def _sc_gather_max(tbl, idx4):
    mesh = plsc.VectorSubcoreMesh(core_axis_name="c", subcore_axis_name="s")

    @functools.partial(
        pl.kernel, mesh=mesh,
        compiler_params=pltpu.CompilerParams(needs_layout_passes=False),
        out_type=jax.ShapeDtypeStruct((2, _W2, _NH), jnp.int32),
        scratch_types=[
            pltpu.VMEM((_WPT, _N_PAD), jnp.int32),
            pltpu.VMEM((2, _K, _BLK), jnp.int32),
            pltpu.VMEM((_WPT, _NH), jnp.int32),
            pltpu.SemaphoreType.DMA,
            pltpu.SemaphoreType.DMA,
        ],
    )
    def k(tbl_hbm, idx_hbm, out_hbm, tbl_v, idx_v, out_v, s0, s1):
        nh = lax.axis_index("c")
        wq = lax.axis_index("s")
        pltpu.sync_copy(tbl_hbm.at[pl.ds(wq * _WPT, _WPT)], tbl_v)
        sems = (s0, s1)
        for b in range(2):
            pltpu.async_copy(idx_hbm.at[nh, b], idx_v.at[b], sems[b])

        def blk_body(b2, carry):
            for b in range(2):
                blk = b2 * 2 + b
                pltpu.make_async_copy(
                    idx_hbm.at[nh, blk], idx_v.at[b], sems[b]).wait()

                def grp_body(g, carry2, b=b, blk=blk):
                    los = [None] * _WPT
                    his = [None] * _WPT
                    for kk in range(_K):
                        vidx = idx_v[b, kk, pl.ds(g * 16, 16)]
                        for w in range(_WPT):
                            wvec = jnp.full((16,), w, jnp.int32)
                            v = plsc.load_gather(tbl_v, [wvec, vidx])
                            if kk == 0:
                                los[w] = v << 16
                                his[w] = v >> 16
                            else:
                                los[w] = jnp.maximum(los[w], v << 16)
                                his[w] = jnp.maximum(his[w], v >> 16)
                    for w in range(_WPT):
                        out_v[w, pl.ds(blk * _BLK + g * 16, 16)] = (
                            (los[w] >> 16) | (his[w] << 16))
                    return carry2

                lax.fori_loop(0, _BLK // 16, grp_body, 0)

                @pl.when(blk + 2 < _NBLK)
                def _(b=b, blk=blk):
                    pltpu.async_copy(
                        idx_hbm.at[nh, blk + 2], idx_v.at[b], sems[b])
            return carry

        lax.fori_loop(0, _NBLK // 2, blk_body, 0)
        pltpu.sync_copy(out_v, out_hbm.at[nh, pl.ds(wq * _WPT, _WPT)])

    return k(tbl, idx4)


def kernel(x, edge_index, W_pre, W_nn, bias):
    B, C_in, N, _ = x.shape
    C_out = W_pre.shape[0]
    assert (B, C_in, C_out, edge_index.shape[-1]) == (1, _C, _C, _K)

    X = x[0, :, :, 0]
    idx = edge_index[0, 0]
    pad = _N_PAD - N
    Xp = jnp.pad(X, ((0, 0), (0, pad)))
    idxp = jnp.pad(idx, ((0, pad), (0, 0)))
    idx4 = idxp.reshape(2, _NBLK, _BLK, _K).transpose(0, 1, 3, 2)

    h = pl.pallas_call(
        _pre_body,
        out_shape=jax.ShapeDtypeStruct((C_out, _N_PAD), jnp.bfloat16),
    )(Xp, W_pre)
    tbl = lax.bitcast_convert_type(
        h.reshape(_W2, 2, _N_PAD).transpose(0, 2, 1), jnp.int32)

    aggr_p = _sc_gather_max(tbl, idx4)
    aggr = lax.bitcast_convert_type(aggr_p, jnp.bfloat16)
    aggr = aggr.transpose(1, 3, 0, 2).reshape(_C, _N_PAD)

    Wx = W_nn[:, :C_in]
    Wa = W_nn[:, C_in:]
    b2 = bias.reshape(C_out, 1)
    o = pl.pallas_call(
        _post_body,
        out_shape=jax.ShapeDtypeStruct((C_out, _N_PAD), jnp.float32),
    )(Xp, aggr, Wx, Wa, b2)

    return o[:, :N].reshape(1, C_out, N, 1)


# speedup vs baseline: 20.1972x; 1.2485x over previous
import functools

import jax
import jax.numpy as jnp
from jax import lax
from jax.experimental import pallas as pl
from jax.experimental.pallas import tpu as pltpu
from jax.experimental.pallas import tpu_sc as plsc

_NC = 2
_NS = 16
_N_PAD = 10240
_NH = _N_PAD // 2
_BLK = 640
_NBLK = _NH // _BLK
_K = 32
_C = 128
_W2 = _C // 2
_WPT = _W2 // _NS


def _pre_body(x_ref, w_ref, o_ref):
    h = lax.dot_general(w_ref[...], x_ref[...], (((1,), (0,)), ((), ())),
                        preferred_element_type=jnp.float32)
    o_ref[...] = jnp.maximum(h, 0.0).astype(jnp.bfloat16)


def _post_body(x_ref, a_ref, wx_ref, wa_ref, b_ref, o_ref):
    a = a_ref[...].astype(jnp.float32)
    t = lax.dot_general(wx_ref[...], x_ref[...], (((1,), (0,)), ((), ())),
                        preferred_element_type=jnp.float32)
    t = t + lax.dot_general(wa_ref[...], a, (((1,), (0,)), ((), ())),
                            preferred_element_type=jnp.float32)
    t = jnp.maximum(t, 0.0) + b_ref[...]
    nrm = jnp.sqrt(jnp.sum(t * t, axis=0, keepdims=True))
    o_ref[...] = t / jnp.maximum(nrm, 1e-12)


def _sc_gather_max(tbl, idx4):
    mesh = plsc.VectorSubcoreMesh(core_axis_name="c", subcore_axis_name="s")

    @functools.partial(
        pl.kernel, mesh=mesh,
        compiler_params=pltpu.CompilerParams(needs_layout_passes=False),
        out_type=jax.ShapeDtypeStruct((2, _W2, _NH), jnp.int32),
        scratch_types=[
            pltpu.VMEM((_WPT, _N_PAD), jnp.int32),
            pltpu.VMEM((2, _K, _BLK), jnp.int32),
            pltpu.VMEM((_WPT, _NH), jnp.int32),
            pltpu.SemaphoreType.DMA,
            pltpu.SemaphoreType.DMA,
        ],
    )
    def k(tbl_hbm, idx_hbm, out_hbm, tbl_v, idx_v, out_v, s0, s1):
        nh = lax.axis_index("c")
        wq = lax.axis_index("s")
        pltpu.sync_copy(tbl_hbm.at[pl.ds(wq * _WPT, _WPT)], tbl_v)
        sems = (s0, s1)
        for b in range(2):
            pltpu.async_copy(idx_hbm.at[nh, b], idx_v.at[b], sems[b])

        def blk_body(b2, carry):
            for b in range(2):
                blk = b2 * 2 + b
                pltpu.make_async_copy(
                    idx_hbm.at[nh, blk], idx_v.at[b], sems[b]).wait()

                def grp_body(g, carry2, b=b, blk=blk):
                    accs = [None] * _WPT
                    for kk in range(_K):
                        vidx = idx_v[b, kk, pl.ds(g * 16, 16)]
                        for w in range(_WPT):
                            wvec = jnp.full((16,), w, jnp.int32)
                            v = plsc.bitcast(
                                plsc.load_gather(tbl_v, [wvec, vidx]),
                                jnp.bfloat16)
                            accs[w] = v if kk == 0 else jnp.maximum(accs[w], v)
                    for w in range(_WPT):
                        out_v[w, pl.ds(blk * _BLK + g * 16, 16)] = (
                            plsc.bitcast(accs[w], jnp.int32))
                    return carry2

                lax.fori_loop(0, _BLK // 16, grp_body, 0)

                @pl.when(blk + 2 < _NBLK)
                def _(b=b, blk=blk):
                    pltpu.async_copy(
                        idx_hbm.at[nh, blk + 2], idx_v.at[b], sems[b])
            return carry

        lax.fori_loop(0, _NBLK // 2, blk_body, 0)
        pltpu.sync_copy(out_v, out_hbm.at[nh, pl.ds(wq * _WPT, _WPT)])

    return k(tbl, idx4)


def kernel(x, edge_index, W_pre, W_nn, bias):
    B, C_in, N, _ = x.shape
    C_out = W_pre.shape[0]
    assert (B, C_in, C_out, edge_index.shape[-1]) == (1, _C, _C, _K)

    X = x[0, :, :, 0]
    idx = edge_index[0, 0]
    pad = _N_PAD - N
    Xp = jnp.pad(X, ((0, 0), (0, pad)))
    idxp = jnp.pad(idx, ((0, pad), (0, 0)))
    idx4 = idxp.reshape(2, _NBLK, _BLK, _K).transpose(0, 1, 3, 2)

    h = pl.pallas_call(
        _pre_body,
        out_shape=jax.ShapeDtypeStruct((C_out, _N_PAD), jnp.bfloat16),
    )(Xp, W_pre)
    tbl = lax.bitcast_convert_type(
        h.reshape(_W2, 2, _N_PAD).transpose(0, 2, 1), jnp.int32)

    aggr_p = _sc_gather_max(tbl, idx4)
    aggr = lax.bitcast_convert_type(aggr_p, jnp.bfloat16)
    aggr = aggr.transpose(1, 3, 0, 2).reshape(_C, _N_PAD)

    Wx = W_nn[:, :C_in]
    Wa = W_nn[:, C_in:]
    b2 = bias.reshape(C_out, 1)
    o = pl.pallas_call(
        _post_body,
        out_shape=jax.ShapeDtypeStruct((C_out, _N_PAD), jnp.float32),
    )(Xp, aggr, Wx, Wa, b2)

    return o[:, :N].reshape(1, C_out, N, 1)


# speedup vs baseline: 24.3761x; 1.2069x over previous
import functools

import jax
import jax.numpy as jnp
from jax import lax
from jax.experimental import pallas as pl
from jax.experimental.pallas import tpu as pltpu
from jax.experimental.pallas import tpu_sc as plsc

_NC = 2
_NS = 16
_N_PAD = 10240
_NH = _N_PAD // 2
_BLK = 640
_NBLK = _NH // _BLK
_K = 32
_C = 128
_W2 = _C // 2
_WPT = _W2 // _NS


def _rne_bf16_bits(h):
    u = lax.bitcast_convert_type(h, jnp.int32)
    return (u + 0x7FFF + ((u >> 16) & 1)) >> 16


def _pre_body(x_ref, wpe_ref, wpo_ref, o_ref):
    x = x_ref[...]
    he = jnp.maximum(
        lax.dot_general(wpe_ref[...], x, (((1,), (0,)), ((), ())),
                        preferred_element_type=jnp.float32), 0.0)
    ho = jnp.maximum(
        lax.dot_general(wpo_ref[...], x, (((1,), (0,)), ((), ())),
                        preferred_element_type=jnp.float32), 0.0)
    o_ref[...] = (_rne_bf16_bits(ho) << 16) | _rne_bf16_bits(he)


def _post_body(x_ref, ap_ref, wx_ref, wae_ref, wao_ref, b_ref, o_ref):
    for h in range(2):
        a = ap_ref[h]
        lof = lax.bitcast_convert_type(a << 16, jnp.float32)
        hif = lax.bitcast_convert_type(a & jnp.int32(-65536), jnp.float32)
        t = lax.dot_general(wx_ref[...], x_ref[:, pl.ds(h * _NH, _NH)],
                            (((1,), (0,)), ((), ())),
                            preferred_element_type=jnp.float32)
        t = t + lax.dot_general(wae_ref[...], lof, (((1,), (0,)), ((), ())),
                                preferred_element_type=jnp.float32)
        t = t + lax.dot_general(wao_ref[...], hif, (((1,), (0,)), ((), ())),
                                preferred_element_type=jnp.float32)
        t = jnp.maximum(t, 0.0) + b_ref[...]
        nrm = jnp.sqrt(jnp.sum(t * t, axis=0, keepdims=True))
        o_ref[:, pl.ds(h * _NH, _NH)] = t / jnp.maximum(nrm, 1e-12)


def _sc_gather_max(tbl, idx4):
    mesh = plsc.VectorSubcoreMesh(core_axis_name="c", subcore_axis_name="s")

    @functools.partial(
        pl.kernel, mesh=mesh,
        compiler_params=pltpu.CompilerParams(needs_layout_passes=False),
        out_type=jax.ShapeDtypeStruct((2, _W2, _NH), jnp.int32),
        scratch_types=[
            pltpu.VMEM((_WPT, _N_PAD), jnp.int32),
            pltpu.VMEM((2, _K, _BLK), jnp.int32),
            pltpu.VMEM((_WPT, _NH), jnp.int32),
            pltpu.SemaphoreType.DMA,
            pltpu.SemaphoreType.DMA,
        ],
    )
    def k(tbl_hbm, idx_hbm, out_hbm, tbl_v, idx_v, out_v, s0, s1):
        nh = lax.axis_index("c")
        wq = lax.axis_index("s")
        pltpu.sync_copy(tbl_hbm.at[pl.ds(wq * _WPT, _WPT)], tbl_v)
        sems = (s0, s1)
        for b in range(2):
            pltpu.async_copy(idx_hbm.at[nh, b], idx_v.at[b], sems[b])

        def blk_body(b2, carry):
            for b in range(2):
                blk = b2 * 2 + b
                pltpu.make_async_copy(
                    idx_hbm.at[nh, blk], idx_v.at[b], sems[b]).wait()

                def grp_body(g, carry2, b=b, blk=blk):
                    accs = [None] * _WPT
                    for kk in range(_K):
                        vidx = idx_v[b, kk, pl.ds(g * 16, 16)]
                        for w in range(_WPT):
                            wvec = jnp.full((16,), w, jnp.int32)
                            v = plsc.bitcast(
                                plsc.load_gather(tbl_v, [wvec, vidx]),
                                jnp.bfloat16)
                            accs[w] = v if kk == 0 else jnp.maximum(accs[w], v)
                    for w in range(_WPT):
                        out_v[w, pl.ds(blk * _BLK + g * 16, 16)] = (
                            plsc.bitcast(accs[w], jnp.int32))
                    return carry2

                lax.fori_loop(0, _BLK // 16, grp_body, 0)

                @pl.when(blk + 2 < _NBLK)
                def _(b=b, blk=blk):
                    pltpu.async_copy(
                        idx_hbm.at[nh, blk + 2], idx_v.at[b], sems[b])
            return carry

        lax.fori_loop(0, _NBLK // 2, blk_body, 0)
        pltpu.sync_copy(out_v, out_hbm.at[nh, pl.ds(wq * _WPT, _WPT)])

    return k(tbl, idx4)


def kernel(x, edge_index, W_pre, W_nn, bias):
    B, C_in, N, _ = x.shape
    C_out = W_pre.shape[0]
    assert (B, C_in, C_out, edge_index.shape[-1]) == (1, _C, _C, _K)

    X = x[0, :, :, 0]
    idx = edge_index[0, 0]
    pad = _N_PAD - N
    Xp = jnp.pad(X, ((0, 0), (0, pad)))
    idxp = jnp.pad(idx, ((0, pad), (0, 0)))
    idx4 = idxp.reshape(2, _NBLK, _BLK, _K).transpose(0, 1, 3, 2)

    tbl = pl.pallas_call(
        _pre_body,
        out_shape=jax.ShapeDtypeStruct((_W2, _N_PAD), jnp.int32),
    )(Xp, W_pre[0::2], W_pre[1::2])

    aggr_p = _sc_gather_max(tbl, idx4)

    Wx = W_nn[:, :C_in]
    Wa = W_nn[:, C_in:]
    b2 = bias.reshape(C_out, 1)
    o = pl.pallas_call(
        _post_body,
        out_shape=jax.ShapeDtypeStruct((C_out, _N_PAD), jnp.float32),
    )(Xp, aggr_p, Wx, Wa[:, 0::2], Wa[:, 1::2], b2)

    return o[:, :N].reshape(1, C_out, N, 1)


# speedup vs baseline: 25.1267x; 1.0308x over previous
import functools

import jax
import jax.numpy as jnp
from jax import lax
from jax.experimental import pallas as pl
from jax.experimental.pallas import tpu as pltpu
from jax.experimental.pallas import tpu_sc as plsc

_NC = 2
_NS = 16
_N_PAD = 10240
_NH = _N_PAD // 2
_BLK = 640
_NBLK = _NH // _BLK
_K = 32
_C = 128
_W2 = _C // 2
_WPT = _W2 // _NS


def _rne_bf16_bits(h):
    u = lax.bitcast_convert_type(h, jnp.int32)
    return (u + 0x7FFF + ((u >> 16) & 1)) >> 16


def _pre_body(x_ref, wp_ref, o_ref):
    x = x_ref[...]
    hl = jnp.maximum(
        lax.dot_general(wp_ref[pl.ds(0, _W2), :], x, (((1,), (0,)), ((), ())),
                        preferred_element_type=jnp.float32), 0.0)
    hh = jnp.maximum(
        lax.dot_general(wp_ref[pl.ds(_W2, _W2), :], x,
                        (((1,), (0,)), ((), ())),
                        preferred_element_type=jnp.float32), 0.0)
    o_ref[...] = (_rne_bf16_bits(hh) << 16) | _rne_bf16_bits(hl)


def _post_body(x_ref, ap_ref, wn_ref, b_ref, o_ref):
    wx = wn_ref[:, pl.ds(0, _C)]
    wal = wn_ref[:, pl.ds(_C, _W2)]
    wah = wn_ref[:, pl.ds(_C + _W2, _W2)]
    for h in range(2):
        a = ap_ref[h]
        lof = lax.bitcast_convert_type(a << 16, jnp.float32)
        hif = lax.bitcast_convert_type(a & jnp.int32(-65536), jnp.float32)
        t = lax.dot_general(wx, x_ref[:, pl.ds(h * _NH, _NH)],
                            (((1,), (0,)), ((), ())),
                            preferred_element_type=jnp.float32)
        t = t + lax.dot_general(wal, lof, (((1,), (0,)), ((), ())),
                                preferred_element_type=jnp.float32)
        t = t + lax.dot_general(wah, hif, (((1,), (0,)), ((), ())),
                                preferred_element_type=jnp.float32)
        t = jnp.maximum(t, 0.0) + b_ref[...]
        nrm = jnp.sqrt(jnp.sum(t * t, axis=0, keepdims=True))
        o_ref[:, pl.ds(h * _NH, _NH)] = t / jnp.maximum(nrm, 1e-12)


def _sc_gather_max(tbl, idx4):
    mesh = plsc.VectorSubcoreMesh(core_axis_name="c", subcore_axis_name="s")

    @functools.partial(
        pl.kernel, mesh=mesh,
        compiler_params=pltpu.CompilerParams(needs_layout_passes=False),
        out_type=jax.ShapeDtypeStruct((2, _W2, _NH), jnp.int32),
        scratch_types=[
            pltpu.VMEM((_WPT, _N_PAD), jnp.int32),
            pltpu.VMEM((2, _K, _BLK), jnp.int32),
            pltpu.VMEM((_WPT, _NH), jnp.int32),
            pltpu.SemaphoreType.DMA,
            pltpu.SemaphoreType.DMA,
        ],
    )
    def k(tbl_hbm, idx_hbm, out_hbm, tbl_v, idx_v, out_v, s0, s1):
        nh = lax.axis_index("c")
        wq = lax.axis_index("s")
        pltpu.sync_copy(tbl_hbm.at[pl.ds(wq * _WPT, _WPT)], tbl_v)
        sems = (s0, s1)
        for b in range(2):
            pltpu.async_copy(idx_hbm.at[nh, b], idx_v.at[b], sems[b])

        def blk_body(b2, carry):
            for b in range(2):
                blk = b2 * 2 + b
                pltpu.make_async_copy(
                    idx_hbm.at[nh, blk], idx_v.at[b], sems[b]).wait()

                def grp_body(g, carry2, b=b, blk=blk):
                    accs = [None] * _WPT
                    for kk in range(_K):
                        vidx = idx_v[b, kk, pl.ds(g * 16, 16)]
                        for w in range(_WPT):
                            wvec = jnp.full((16,), w, jnp.int32)
                            v = plsc.bitcast(
                                plsc.load_gather(tbl_v, [wvec, vidx]),
                                jnp.bfloat16)
                            accs[w] = v if kk == 0 else jnp.maximum(accs[w], v)
                    for w in range(_WPT):
                        out_v[w, pl.ds(blk * _BLK + g * 16, 16)] = (
                            plsc.bitcast(accs[w], jnp.int32))
                    return carry2

                lax.fori_loop(0, _BLK // 16, grp_body, 0)

                @pl.when(blk + 2 < _NBLK)
                def _(b=b, blk=blk):
                    pltpu.async_copy(
                        idx_hbm.at[nh, blk + 2], idx_v.at[b], sems[b])
            return carry

        lax.fori_loop(0, _NBLK // 2, blk_body, 0)
        pltpu.sync_copy(out_v, out_hbm.at[nh, pl.ds(wq * _WPT, _WPT)])

    return k(tbl, idx4)


def kernel(x, edge_index, W_pre, W_nn, bias):
    B, C_in, N, _ = x.shape
    C_out = W_pre.shape[0]
    assert (B, C_in, C_out, edge_index.shape[-1]) == (1, _C, _C, _K)

    X = x[0, :, :, 0]
    idx = edge_index[0, 0]
    pad = _N_PAD - N
    Xp = jnp.pad(X, ((0, 0), (0, pad)))
    idxp = jnp.pad(idx, ((0, pad), (0, 0)))
    idx4 = idxp.reshape(2, _NBLK, _BLK, _K).transpose(0, 1, 3, 2)

    tbl = pl.pallas_call(
        _pre_body,
        out_shape=jax.ShapeDtypeStruct((_W2, _N_PAD), jnp.int32),
    )(Xp, W_pre)

    aggr_p = _sc_gather_max(tbl, idx4)

    o = pl.pallas_call(
        _post_body,
        out_shape=jax.ShapeDtypeStruct((C_out, _N_PAD), jnp.float32),
    )(Xp, aggr_p, W_nn, bias.reshape(C_out, 1))

    return o[:, :N].reshape(1, C_out, N, 1)
